# baseline (speedup 1.0000x reference)
import functools

import jax
import jax.numpy as jnp
from jax import lax
from jax.experimental import pallas as pl
from jax.experimental.pallas import tpu as pltpu
from jax.experimental.pallas import tpu_sc as plsc

NC = 2
NS = 16
NW = NC * NS


def _ln(h, g, be):
    mu = jnp.mean(h, axis=-1, keepdims=True)
    var = jnp.mean((h - mu) * (h - mu), axis=-1, keepdims=True)
    return (h - mu) * lax.rsqrt(var + 1e-5) * g + be


def _ln_mm(h, g, be, M):
    mu = jnp.dot(h, M, preferred_element_type=jnp.float32)
    d = h - mu
    var = jnp.dot(d * d, M, preferred_element_type=jnp.float32)
    return d * lax.rsqrt(var + 1e-5) * g + be


def _enc_node_body(x, W1, b1, W2, b2, g, be, Ws, Wr, hb1, nl_o, ps_o, pr_o):
    h = jnp.maximum(jnp.dot(x[...], W1[...], preferred_element_type=jnp.float32) + b1[...], 0.0)
    h = jnp.maximum(jnp.dot(h, W2[...], preferred_element_type=jnp.float32) + b2[...], 0.0)
    nl = _ln(h, g[...], be[...])
    nl_o[...] = nl
    ps_o[...] = (jnp.dot(nl, Ws[...], preferred_element_type=jnp.float32)
                 + hb1[...]).astype(jnp.bfloat16)
    pr_o[...] = (jnp.dot(nl, Wr[...], preferred_element_type=jnp.float32)
                 + hb1[...]).astype(jnp.bfloat16)


def _enc_edge_body(x, W1, b1, W2, b2, g, be, M, el_o):
    h = jnp.maximum(jnp.dot(x[...], W1[...], preferred_element_type=jnp.float32) + b1[...], 0.0)
    h = jnp.maximum(jnp.dot(h, W2[...], preferred_element_type=jnp.float32) + b2[...], 0.0)
    el_o[...] = _ln_mm(h, g[...], be[...], M[...])


def _edge_body(gs, gr, el, We, W2, b2, g, be, M, ne_o, el_o):
    elf = el[...]
    pre1 = (gs[...].astype(jnp.float32) + gr[...].astype(jnp.float32)
            + jnp.dot(elf, We[...], preferred_element_type=jnp.float32))
    h = jnp.maximum(pre1, 0.0)
    h = jnp.maximum(jnp.dot(h, W2[...], preferred_element_type=jnp.float32) + b2[...], 0.0)
    ne = _ln_mm(h, g[...], be[...], M[...])
    ne_o[...] = ne
    if el_o is not None:
        el_o[...] = elf + ne


def _node_body(nl, a0, a1, W1a, W1b, b1, W2, b2, g, be, Ws, Wr, hb1,
               nl_o, ps_o, pr_o):
    aggr = a0[...] + a1[...]
    pre1 = (jnp.dot(nl[...], W1a[...], preferred_element_type=jnp.float32)
            + jnp.dot(aggr, W1b[...], preferred_element_type=jnp.float32) + b1[...])
    h = jnp.maximum(pre1, 0.0)
    h = jnp.maximum(jnp.dot(h, W2[...], preferred_element_type=jnp.float32) + b2[...], 0.0)
    out = _ln(h, g[...], be[...]) + nl[...]
    nl_o[...] = out
    if ps_o is not None:
        ps_o[...] = (jnp.dot(out, Ws[...], preferred_element_type=jnp.float32)
                     + hb1[...]).astype(jnp.bfloat16)
        pr_o[...] = (jnp.dot(out, Wr[...], preferred_element_type=jnp.float32)
                     + hb1[...]).astype(jnp.bfloat16)


def _dec_body(nl, W1, b1, W2s, b2s, out_o):
    h = jnp.dot(nl[...], W1[...], preferred_element_type=jnp.float32) + b1[...]
    h = h / (1.0 + jnp.exp(-h))
    z = jnp.dot(h, W2s[...], preferred_element_type=jnp.float32) + b2s[...]
    for t in range(z.shape[-1]):
        out_o[t] = z[:, t:t + 1]


def _row_spec(B, D):
    return pl.BlockSpec((B, D), lambda i: (i, 0))


def _full_spec(shape):
    return pl.BlockSpec(shape, lambda i: tuple(0 for _ in shape))


def _make_sc_gather(E, Dl, CH):
    EPW = E // NW
    NCH = EPW // CH
    mesh = plsc.VectorSubcoreMesh(core_axis_name="c", subcore_axis_name="s",
                                  num_cores=NC, num_subcores=NS)

    @functools.partial(
        pl.kernel,
        out_type=(jax.ShapeDtypeStruct((E, Dl), jnp.bfloat16),
                  jax.ShapeDtypeStruct((E, Dl), jnp.bfloat16)),
        mesh=mesh,
        compiler_params=pltpu.CompilerParams(use_tc_tiling_on_sc=False),
        scratch_types=[
            pltpu.VMEM((NCH, CH), jnp.int32),
            pltpu.VMEM((NCH, CH), jnp.int32),
            pltpu.VMEM((2, CH, Dl), jnp.bfloat16),
            pltpu.VMEM((2, CH, Dl), jnp.bfloat16),
            pltpu.SemaphoreType.DMA,
            pltpu.SemaphoreType.DMA,
        ],
    )
    def gather_k(ps_h, pr_h, snd3_h, rcv3_h, gs_h, gr_h,
                 idx_s, idx_r, rows_s, rows_r, sem_g, sem_w):
        wid = lax.axis_index("s") * NC + lax.axis_index("c")
        base = wid * EPW
        pltpu.sync_copy(snd3_h.at[wid], idx_s)
        pltpu.sync_copy(rcv3_h.at[wid], idx_r)

        def start_gather(j, slot):
            pltpu.async_copy(ps_h.at[idx_s.at[j]], rows_s.at[slot], sem_g)
            pltpu.async_copy(pr_h.at[idx_r.at[j]], rows_r.at[slot], sem_g)

        def drain2(sem):
            pltpu.make_async_copy(gs_h.at[pl.ds(0, CH)], rows_s.at[0], sem).wait()
            pltpu.make_async_copy(gs_h.at[pl.ds(0, CH)], rows_r.at[0], sem).wait()

        start_gather(0, 0)

        def step(j, carry):
            slot = lax.rem(j, 2)
            drain2(sem_g)

            @pl.when(j > 0)
            def _():
                drain2(sem_w)

            @pl.when(j < NCH - 1)
            def _():
                start_gather(j + 1, 1 - slot)
            off = base + j * CH
            pltpu.async_copy(rows_s.at[slot], gs_h.at[pl.ds(off, CH)], sem_w)
            pltpu.async_copy(rows_r.at[slot], gr_h.at[pl.ds(off, CH)], sem_w)
            return carry

        lax.fori_loop(0, NCH, step, 0)
        drain2(sem_w)

    return gather_k


def _make_sc_scatter(E, Nn, Dl, CH):
    EPW = E // NW
    NCH = EPW // CH
    NPT = Nn // NS
    mesh = plsc.VectorSubcoreMesh(core_axis_name="c", subcore_axis_name="s",
                                  num_cores=NC, num_subcores=NS)

    @functools.partial(
        pl.kernel,
        out_type=jax.ShapeDtypeStruct((2 * Nn, Dl), jnp.float32),
        mesh=mesh,
        compiler_params=pltpu.CompilerParams(use_tc_tiling_on_sc=False),
        scratch_types=[
            pltpu.VMEM((NCH, CH), jnp.int32),
            pltpu.VMEM((2, CH, Dl), jnp.float32),
            pltpu.VMEM_SHARED((Nn, Dl), jnp.float32),
            pltpu.SemaphoreType.DMA,
        ],
    )
    def scatter_k(ne_h, rcv3_h, zeros_h, out_h, idx_v, rows_v, aggr_sh, sem_l):
        c = lax.axis_index("c")
        s = lax.axis_index("s")
        wid = s * NC + c
        base = wid * EPW
        pltpu.sync_copy(rcv3_h.at[wid], idx_v)
        pltpu.sync_copy(zeros_h.at[pl.ds(s * NPT, NPT)],
                        aggr_sh.at[pl.ds(s * NPT, NPT)])
        plsc.subcore_barrier()

        pltpu.async_copy(ne_h.at[pl.ds(base, CH)], rows_v.at[0], sem_l)

        def step(j, carry):
            slot = lax.rem(j, 2)
            pltpu.make_async_copy(ne_h.at[pl.ds(0, CH)], rows_v.at[0],
                                  sem_l).wait()

            @pl.when(j < NCH - 1)
            def _():
                pltpu.async_copy(ne_h.at[pl.ds(base + (j + 1) * CH, CH)],
                                 rows_v.at[1 - slot], sem_l)
            pltpu.sync_copy(rows_v.at[slot], aggr_sh.at[idx_v.at[j]], add=True)
            return carry

        lax.fori_loop(0, NCH, step, 0)
        plsc.subcore_barrier()

        pltpu.sync_copy(aggr_sh.at[pl.ds(s * NPT, NPT)],
                        out_h.at[pl.ds(c * Nn + s * NPT, NPT)])

    return scatter_k


def kernel(node_features, mesh_edge_features, en_W1, en_b1, en_W2, en_b2, en_g,
           en_be, ee_W1, ee_b1, ee_W2, ee_b2, ee_g, ee_be, be_W1, be_b1, be_W2,
           be_b2, be_g, be_be, bn_W1, bn_b1, bn_W2, bn_b2, bn_g, bn_be, dc_W1,
           dc_b1, dc_W2, dc_b2, senders, receivers):
    f32 = jnp.float32
    Nn = node_features.shape[1]
    E = mesh_edge_features.shape[1]
    Dl = en_W2.shape[1]
    MP = be_W1.shape[0]
    TW = dc_W2.shape[1]

    BN = 2000
    BE = 8000
    CH = 400
    Mavg = jnp.full((Dl, Dl), 1.0 / Dl, f32)

    nf = node_features[0]
    ef = mesh_edge_features[0]
    r2 = lambda v: v.reshape(1, -1)

    sc_gather = _make_sc_gather(E, Dl, CH)
    sc_scatter = _make_sc_scatter(E, Nn, Dl, CH)
    zeros_n = jnp.zeros((Nn, Dl), f32)
    NCH = E // NW // CH
    snd3 = senders.reshape(NW, NCH, CH)
    rcv3 = receivers.reshape(NW, NCH, CH)

    Ws0, Wr0, _ = jnp.split(be_W1[0], 3, axis=0)
    hb1_0 = 0.5 * r2(be_b1[0])
    nl, ps, pr = pl.pallas_call(
        _enc_node_body,
        grid=(Nn // BN,),
        in_specs=[_row_spec(BN, nf.shape[1]), _full_spec(en_W1.shape),
                  _full_spec((1, Dl)), _full_spec(en_W2.shape),
                  _full_spec((1, Dl)), _full_spec((1, Dl)), _full_spec((1, Dl)),
                  _full_spec((Dl, Dl)), _full_spec((Dl, Dl)), _full_spec((1, Dl))],
        out_specs=[_row_spec(BN, Dl)] * 3,
        out_shape=[jax.ShapeDtypeStruct((Nn, Dl), f32),
                   jax.ShapeDtypeStruct((Nn, Dl), jnp.bfloat16),
                   jax.ShapeDtypeStruct((Nn, Dl), jnp.bfloat16)],
    )(nf, en_W1, r2(en_b1), en_W2, r2(en_b2), r2(en_g), r2(en_be),
      Ws0, Wr0, hb1_0)

    el = pl.pallas_call(
        _enc_edge_body,
        grid=(E // BE,),
        in_specs=[_row_spec(BE, ef.shape[1]), _full_spec(ee_W1.shape),
                  _full_spec((1, Dl)), _full_spec(ee_W2.shape),
                  _full_spec((1, Dl)), _full_spec((1, Dl)), _full_spec((1, Dl)),
                  _full_spec((Dl, Dl))],
        out_specs=_row_spec(BE, Dl),
        out_shape=jax.ShapeDtypeStruct((E, Dl), f32),
    )(ef, ee_W1, r2(ee_b1), ee_W2, r2(ee_b2), r2(ee_g), r2(ee_be), Mavg)

    for i in range(MP):
        gs, gr = sc_gather(ps, pr, snd3, rcv3)

        _, _, We = jnp.split(be_W1[i], 3, axis=0)
        last = i == MP - 1
        edge_body = _edge_body if not last else (
            lambda gs_, gr_, el_, We_, W2_, b2_, g_, be_, M_, ne_o:
            _edge_body(gs_, gr_, el_, We_, W2_, b2_, g_, be_, M_, ne_o, None))
        out_specs = [_row_spec(BE, Dl)] * (1 if last else 2)
        out_shape = [jax.ShapeDtypeStruct((E, Dl), f32)]
        if not last:
            out_shape.append(jax.ShapeDtypeStruct((E, Dl), f32))
        res = pl.pallas_call(
            edge_body,
            grid=(E // BE,),
            in_specs=[_row_spec(BE, Dl)] * 3 + [
                _full_spec((Dl, Dl)), _full_spec((Dl, Dl)),
                _full_spec((1, Dl)), _full_spec((1, Dl)), _full_spec((1, Dl)),
                _full_spec((Dl, Dl))],
            out_specs=out_specs,
            out_shape=out_shape,
        )(gs, gr, el, We, be_W2[i], r2(be_b2[i]), r2(be_g[i]), r2(be_be[i]), Mavg)
        if last:
            ne = res[0]
        else:
            ne, el = res

        agg2 = sc_scatter(ne, rcv3, zeros_n)
        a0, a1 = agg2[:Nn], agg2[Nn:]

        W1a, W1b = jnp.split(bn_W1[i], 2, axis=0)
        if not last:
            Wsn, Wrn, _ = jnp.split(be_W1[i + 1], 3, axis=0)
            hb1_n = 0.5 * r2(be_b1[i + 1])
            nl, ps, pr = pl.pallas_call(
                _node_body,
                grid=(Nn // BN,),
                in_specs=[_row_spec(BN, Dl)] * 3 + [
                    _full_spec((Dl, Dl)), _full_spec((Dl, Dl)),
                    _full_spec((1, Dl)), _full_spec((Dl, Dl)),
                    _full_spec((1, Dl)), _full_spec((1, Dl)), _full_spec((1, Dl)),
                    _full_spec((Dl, Dl)), _full_spec((Dl, Dl)), _full_spec((1, Dl))],
                out_specs=[_row_spec(BN, Dl)] * 3,
                out_shape=[jax.ShapeDtypeStruct((Nn, Dl), f32),
                           jax.ShapeDtypeStruct((Nn, Dl), jnp.bfloat16),
                           jax.ShapeDtypeStruct((Nn, Dl), jnp.bfloat16)],
            )(nl, a0, a1, W1a, W1b, r2(bn_b1[i]), bn_W2[i], r2(bn_b2[i]),
              r2(bn_g[i]), r2(bn_be[i]), Wsn, Wrn, hb1_n)
        else:
            node_last = (lambda nl_, a0_, a1_, W1a_, W1b_, b1_, W2_, b2_, g_, be_, nl_o:
                         _node_body(nl_, a0_, a1_, W1a_, W1b_, b1_, W2_, b2_,
                                    g_, be_, None, None, None, nl_o, None, None))
            nl = pl.pallas_call(
                node_last,
                grid=(Nn // BN,),
                in_specs=[_row_spec(BN, Dl)] * 3 + [
                    _full_spec((Dl, Dl)), _full_spec((Dl, Dl)),
                    _full_spec((1, Dl)), _full_spec((Dl, Dl)),
                    _full_spec((1, Dl)), _full_spec((1, Dl)), _full_spec((1, Dl))],
                out_specs=_row_spec(BN, Dl),
                out_shape=jax.ShapeDtypeStruct((Nn, Dl), f32),
            )(nl, a0, a1, W1a, W1b, r2(bn_b1[i]), bn_W2[i], r2(bn_b2[i]),
              r2(bn_g[i]), r2(bn_be[i]))

    dt = jnp.arange(1, TW + 1, dtype=f32)
    W2s = dc_W2 * dt[None, :]
    b2s = r2(dc_b2 * dt)
    out = pl.pallas_call(
        _dec_body,
        grid=(Nn // BN,),
        in_specs=[_row_spec(BN, Dl), _full_spec(dc_W1.shape),
                  _full_spec((1, dc_W1.shape[1])), _full_spec(dc_W2.shape),
                  _full_spec((1, TW))],
        out_specs=pl.BlockSpec((TW, BN, 1), lambda i: (0, i, 0)),
        out_shape=jax.ShapeDtypeStruct((TW, Nn, 1), f32),
    )(nl, dc_W1, r2(dc_b1), W2s, b2s)
    return out


# speedup vs baseline: 2.2891x; 2.2891x over previous
import functools

import jax
import jax.numpy as jnp
from jax import lax
from jax.experimental import pallas as pl
from jax.experimental.pallas import tpu as pltpu
from jax.experimental.pallas import tpu_sc as plsc

NC = 2
NS = 16
NW = NC * NS


def _ln(h, g, be):
    mu = jnp.mean(h, axis=-1, keepdims=True)
    var = jnp.mean((h - mu) * (h - mu), axis=-1, keepdims=True)
    return (h - mu) * lax.rsqrt(var + 1e-5) * g + be


def _ln_mm(h, g, be, M):
    mu = jnp.dot(h, M, preferred_element_type=jnp.float32)
    d = h - mu
    var = jnp.dot(d * d, M, preferred_element_type=jnp.float32)
    return d * lax.rsqrt(var + 1e-5) * g + be


def _enc_node_body(x, W1, b1, W2, b2, g, be, Ws, Wr, hb1, nl_o, ps_o, pr_o):
    h = jnp.maximum(jnp.dot(x[...], W1[...], preferred_element_type=jnp.float32) + b1[...], 0.0)
    h = jnp.maximum(jnp.dot(h, W2[...], preferred_element_type=jnp.float32) + b2[...], 0.0)
    nl = _ln(h, g[...], be[...])
    nl_o[...] = nl
    ps_o[...] = jnp.dot(nl, Ws[...], preferred_element_type=jnp.float32) + hb1[...]
    pr_o[...] = jnp.dot(nl, Wr[...], preferred_element_type=jnp.float32) + hb1[...]


def _enc_edge_body(x, W1, b1, W2, b2, g, be, M, el_o):
    h = jnp.maximum(jnp.dot(x[...], W1[...], preferred_element_type=jnp.float32) + b1[...], 0.0)
    h = jnp.maximum(jnp.dot(h, W2[...], preferred_element_type=jnp.float32) + b2[...], 0.0)
    el_o[...] = _ln_mm(h, g[...], be[...], M[...])


def _edge_body(gs, gr, el, We, W2, b2, g, be, M, ne_o, el_o):
    elf = el[...]
    pre1 = (gs[...] + gr[...]
            + jnp.dot(elf, We[...], preferred_element_type=jnp.float32))
    h = jnp.maximum(pre1, 0.0)
    h = jnp.maximum(jnp.dot(h, W2[...], preferred_element_type=jnp.float32) + b2[...], 0.0)
    ne = _ln_mm(h, g[...], be[...], M[...])
    ne_o[...] = ne
    if el_o is not None:
        el_o[...] = elf + ne


def _node_body(nl, a0, a1, W1a, W1b, b1, W2, b2, g, be, Ws, Wr, hb1,
               nl_o, ps_o, pr_o):
    aggr = a0[...] + a1[...]
    pre1 = (jnp.dot(nl[...], W1a[...], preferred_element_type=jnp.float32)
            + jnp.dot(aggr, W1b[...], preferred_element_type=jnp.float32) + b1[...])
    h = jnp.maximum(pre1, 0.0)
    h = jnp.maximum(jnp.dot(h, W2[...], preferred_element_type=jnp.float32) + b2[...], 0.0)
    out = _ln(h, g[...], be[...]) + nl[...]
    nl_o[...] = out
    if ps_o is not None:
        ps_o[...] = jnp.dot(out, Ws[...], preferred_element_type=jnp.float32) + hb1[...]
        pr_o[...] = jnp.dot(out, Wr[...], preferred_element_type=jnp.float32) + hb1[...]


def _dec_body(nl, W1, b1, W2s, b2s, out_o):
    h = jnp.dot(nl[...], W1[...], preferred_element_type=jnp.float32) + b1[...]
    h = h / (1.0 + jnp.exp(-h))
    z = jnp.dot(h, W2s[...], preferred_element_type=jnp.float32) + b2s[...]
    for t in range(z.shape[-1]):
        out_o[t] = z[:, t:t + 1]


def _row_spec(B, D):
    return pl.BlockSpec((B, D), lambda i: (i, 0))


def _full_spec(shape):
    return pl.BlockSpec(shape, lambda i: tuple(0 for _ in shape))


def _make_sc_gather(E, Dl, CH):
    EPW = E // NW
    NCH = EPW // CH
    mesh = plsc.VectorSubcoreMesh(core_axis_name="c", subcore_axis_name="s",
                                  num_cores=NC, num_subcores=NS)

    @functools.partial(
        pl.kernel,
        out_type=(jax.ShapeDtypeStruct((E, Dl), jnp.float32),
                  jax.ShapeDtypeStruct((E, Dl), jnp.float32)),
        mesh=mesh,
        compiler_params=pltpu.CompilerParams(use_tc_tiling_on_sc=False),
        scratch_types=[
            pltpu.VMEM((NCH, CH), jnp.int32),
            pltpu.VMEM((NCH, CH), jnp.int32),
            pltpu.VMEM((2, CH, Dl), jnp.float32),
            pltpu.VMEM((2, CH, Dl), jnp.float32),
            pltpu.SemaphoreType.DMA,
            pltpu.SemaphoreType.DMA,
        ],
    )
    def gather_k(ps_h, pr_h, snd3_h, rcv3_h, gs_h, gr_h,
                 idx_s, idx_r, rows_s, rows_r, sem_g, sem_w):
        wid = lax.axis_index("s") * NC + lax.axis_index("c")
        base = wid * EPW
        pltpu.sync_copy(snd3_h.at[wid], idx_s)
        pltpu.sync_copy(rcv3_h.at[wid], idx_r)

        def start_gather(j, slot):
            pltpu.async_copy(ps_h.at[idx_s.at[j]], rows_s.at[slot], sem_g)
            pltpu.async_copy(pr_h.at[idx_r.at[j]], rows_r.at[slot], sem_g)

        def drain2(sem):
            pltpu.make_async_copy(gs_h.at[pl.ds(0, CH)], rows_s.at[0], sem).wait()
            pltpu.make_async_copy(gs_h.at[pl.ds(0, CH)], rows_r.at[0], sem).wait()

        start_gather(0, 0)

        def step(j, carry):
            slot = lax.rem(j, 2)
            drain2(sem_g)

            @pl.when(j > 0)
            def _():
                drain2(sem_w)

            @pl.when(j < NCH - 1)
            def _():
                start_gather(j + 1, 1 - slot)
            off = base + j * CH
            pltpu.async_copy(rows_s.at[slot], gs_h.at[pl.ds(off, CH)], sem_w)
            pltpu.async_copy(rows_r.at[slot], gr_h.at[pl.ds(off, CH)], sem_w)
            return carry

        lax.fori_loop(0, NCH, step, 0)
        drain2(sem_w)

    return gather_k


def _make_sc_scatter(E, Nn, Dl, CH):
    EPW = E // NW
    NCH = EPW // CH
    NPT = Nn // NS
    mesh = plsc.VectorSubcoreMesh(core_axis_name="c", subcore_axis_name="s",
                                  num_cores=NC, num_subcores=NS)

    @functools.partial(
        pl.kernel,
        out_type=jax.ShapeDtypeStruct((2 * Nn, Dl), jnp.float32),
        mesh=mesh,
        compiler_params=pltpu.CompilerParams(use_tc_tiling_on_sc=False),
        scratch_types=[
            pltpu.VMEM((NCH, CH), jnp.int32),
            pltpu.VMEM((2, CH, Dl), jnp.float32),
            pltpu.VMEM_SHARED((Nn, Dl), jnp.float32),
            pltpu.SemaphoreType.DMA,
        ],
    )
    def scatter_k(ne_h, rcv3_h, zeros_h, out_h, idx_v, rows_v, aggr_sh, sem_l):
        c = lax.axis_index("c")
        s = lax.axis_index("s")
        wid = s * NC + c
        base = wid * EPW
        pltpu.sync_copy(rcv3_h.at[wid], idx_v)
        pltpu.sync_copy(zeros_h.at[pl.ds(s * NPT, NPT)],
                        aggr_sh.at[pl.ds(s * NPT, NPT)])
        plsc.subcore_barrier()

        pltpu.async_copy(ne_h.at[pl.ds(base, CH)], rows_v.at[0], sem_l)

        def step(j, carry):
            slot = lax.rem(j, 2)
            pltpu.make_async_copy(ne_h.at[pl.ds(0, CH)], rows_v.at[0],
                                  sem_l).wait()

            @pl.when(j < NCH - 1)
            def _():
                pltpu.async_copy(ne_h.at[pl.ds(base + (j + 1) * CH, CH)],
                                 rows_v.at[1 - slot], sem_l)
            pltpu.sync_copy(rows_v.at[slot], aggr_sh.at[idx_v.at[j]], add=True)
            return carry

        lax.fori_loop(0, NCH, step, 0)
        plsc.subcore_barrier()

        pltpu.sync_copy(aggr_sh.at[pl.ds(s * NPT, NPT)],
                        out_h.at[pl.ds(c * Nn + s * NPT, NPT)])

    return scatter_k


def kernel(node_features, mesh_edge_features, en_W1, en_b1, en_W2, en_b2, en_g,
           en_be, ee_W1, ee_b1, ee_W2, ee_b2, ee_g, ee_be, be_W1, be_b1, be_W2,
           be_b2, be_g, be_be, bn_W1, bn_b1, bn_W2, bn_b2, bn_g, bn_be, dc_W1,
           dc_b1, dc_W2, dc_b2, senders, receivers):
    f32 = jnp.float32
    Nn = node_features.shape[1]
    E = mesh_edge_features.shape[1]
    Dl = en_W2.shape[1]
    MP = be_W1.shape[0]
    TW = dc_W2.shape[1]

    BN = 2000
    E2 = E // 2
    BE2 = 2000
    CH = 400

    nf = node_features[0]
    ef = mesh_edge_features[0]
    r2 = lambda v: v.reshape(1, -1)
    pair = lambda v: jnp.concatenate([v, v]).reshape(1, -1)

    def bd(W):
        z = jnp.zeros_like(W)
        return jnp.concatenate(
            [jnp.concatenate([W, z], 1), jnp.concatenate([z, W], 1)], 0)

    Mavg2 = bd(jnp.full((Dl, Dl), 1.0 / Dl, f32))

    sc_gather = _make_sc_gather(E, Dl, CH)
    sc_scatter = _make_sc_scatter(E, Nn, Dl, CH)
    zeros_n = jnp.zeros((Nn, Dl), f32)
    NCH = E // NW // CH
    snd3 = senders.reshape(NW, NCH, CH)
    rcv3 = receivers.reshape(NW, NCH, CH)

    Ws0, Wr0, _ = jnp.split(be_W1[0], 3, axis=0)
    hb1_0 = 0.5 * r2(be_b1[0])
    nl, ps, pr = pl.pallas_call(
        _enc_node_body,
        grid=(Nn // BN,),
        in_specs=[_row_spec(BN, nf.shape[1]), _full_spec(en_W1.shape),
                  _full_spec((1, Dl)), _full_spec(en_W2.shape),
                  _full_spec((1, Dl)), _full_spec((1, Dl)), _full_spec((1, Dl)),
                  _full_spec((Dl, Dl)), _full_spec((Dl, Dl)), _full_spec((1, Dl))],
        out_specs=[_row_spec(BN, Dl)] * 3,
        out_shape=[jax.ShapeDtypeStruct((Nn, Dl), f32)] * 3,
    )(nf, en_W1, r2(en_b1), en_W2, r2(en_b2), r2(en_g), r2(en_be),
      Ws0, Wr0, hb1_0)

    EF = ef.shape[1]
    ef2 = ef.reshape(E2, 2 * EF)
    el = pl.pallas_call(
        _enc_edge_body,
        grid=(E2 // BE2,),
        in_specs=[_row_spec(BE2, 2 * EF), _full_spec((2 * EF, 2 * Dl)),
                  _full_spec((1, 2 * Dl)), _full_spec((2 * Dl, 2 * Dl)),
                  _full_spec((1, 2 * Dl)), _full_spec((1, 2 * Dl)),
                  _full_spec((1, 2 * Dl)), _full_spec((2 * Dl, 2 * Dl))],
        out_specs=_row_spec(BE2, 2 * Dl),
        out_shape=jax.ShapeDtypeStruct((E2, 2 * Dl), f32),
    )(ef2, bd(ee_W1), pair(ee_b1), bd(ee_W2), pair(ee_b2), pair(ee_g),
      pair(ee_be), Mavg2)

    for i in range(MP):
        gs, gr = sc_gather(ps, pr, snd3, rcv3)
        gs2 = gs.reshape(E2, 2 * Dl)
        gr2 = gr.reshape(E2, 2 * Dl)

        _, _, We = jnp.split(be_W1[i], 3, axis=0)
        last = i == MP - 1
        edge_body = _edge_body if not last else (
            lambda gs_, gr_, el_, We_, W2_, b2_, g_, be_, M_, ne_o:
            _edge_body(gs_, gr_, el_, We_, W2_, b2_, g_, be_, M_, ne_o, None))
        out_specs = [_row_spec(BE2, 2 * Dl)] * (1 if last else 2)
        out_shape = [jax.ShapeDtypeStruct((E2, 2 * Dl), f32)]
        if not last:
            out_shape.append(jax.ShapeDtypeStruct((E2, 2 * Dl), f32))
        res = pl.pallas_call(
            edge_body,
            grid=(E2 // BE2,),
            in_specs=[_row_spec(BE2, 2 * Dl)] * 3 + [
                _full_spec((2 * Dl, 2 * Dl)), _full_spec((2 * Dl, 2 * Dl)),
                _full_spec((1, 2 * Dl)), _full_spec((1, 2 * Dl)),
                _full_spec((1, 2 * Dl)), _full_spec((2 * Dl, 2 * Dl))],
            out_specs=out_specs,
            out_shape=out_shape,
        )(gs2, gr2, el, bd(We), bd(be_W2[i]), pair(be_b2[i]), pair(be_g[i]),
          pair(be_be[i]), Mavg2)
        if last:
            ne = res[0]
        else:
            ne, el = res

        agg2 = sc_scatter(ne.reshape(E, Dl), rcv3, zeros_n)
        a0, a1 = agg2[:Nn], agg2[Nn:]

        W1a, W1b = jnp.split(bn_W1[i], 2, axis=0)
        if not last:
            Wsn, Wrn, _ = jnp.split(be_W1[i + 1], 3, axis=0)
            hb1_n = 0.5 * r2(be_b1[i + 1])
            nl, ps, pr = pl.pallas_call(
                _node_body,
                grid=(Nn // BN,),
                in_specs=[_row_spec(BN, Dl)] * 3 + [
                    _full_spec((Dl, Dl)), _full_spec((Dl, Dl)),
                    _full_spec((1, Dl)), _full_spec((Dl, Dl)),
                    _full_spec((1, Dl)), _full_spec((1, Dl)), _full_spec((1, Dl)),
                    _full_spec((Dl, Dl)), _full_spec((Dl, Dl)), _full_spec((1, Dl))],
                out_specs=[_row_spec(BN, Dl)] * 3,
                out_shape=[jax.ShapeDtypeStruct((Nn, Dl), f32)] * 3,
            )(nl, a0, a1, W1a, W1b, r2(bn_b1[i]), bn_W2[i], r2(bn_b2[i]),
              r2(bn_g[i]), r2(bn_be[i]), Wsn, Wrn, hb1_n)
        else:
            node_last = (lambda nl_, a0_, a1_, W1a_, W1b_, b1_, W2_, b2_, g_, be_, nl_o:
                         _node_body(nl_, a0_, a1_, W1a_, W1b_, b1_, W2_, b2_,
                                    g_, be_, None, None, None, nl_o, None, None))
            nl = pl.pallas_call(
                node_last,
                grid=(Nn // BN,),
                in_specs=[_row_spec(BN, Dl)] * 3 + [
                    _full_spec((Dl, Dl)), _full_spec((Dl, Dl)),
                    _full_spec((1, Dl)), _full_spec((Dl, Dl)),
                    _full_spec((1, Dl)), _full_spec((1, Dl)), _full_spec((1, Dl))],
                out_specs=_row_spec(BN, Dl),
                out_shape=jax.ShapeDtypeStruct((Nn, Dl), f32),
            )(nl, a0, a1, W1a, W1b, r2(bn_b1[i]), bn_W2[i], r2(bn_b2[i]),
              r2(bn_g[i]), r2(bn_be[i]))

    dt = jnp.arange(1, TW + 1, dtype=f32)
    W2s = dc_W2 * dt[None, :]
    b2s = r2(dc_b2 * dt)
    out = pl.pallas_call(
        _dec_body,
        grid=(Nn // BN,),
        in_specs=[_row_spec(BN, Dl), _full_spec(dc_W1.shape),
                  _full_spec((1, dc_W1.shape[1])), _full_spec(dc_W2.shape),
                  _full_spec((1, TW))],
        out_specs=pl.BlockSpec((TW, BN, 1), lambda i: (0, i, 0)),
        out_shape=jax.ShapeDtypeStruct((TW, Nn, 1), f32),
    )(nl, dc_W1, r2(dc_b1), W2s, b2s)
    return out
